# baseline (speedup 1.0000x reference)
import jax
import jax.numpy as jnp
from jax import lax
from jax.experimental import pallas as pl
from jax.experimental.pallas import tpu as pltpu
from jax.experimental.pallas import tpu_sc as plsc

N_NODES = 50000
NP = N_NODES + 8
CHUNK = 128
BCH = 8
BLOCK = CHUNK * BCH
NTILES = 32
CPT = 392
NBLK = CPT // BCH
EP = NTILES * CPT * CHUNK
BLKN = 2000

_f32 = jnp.float32
_i32 = jnp.int32


def _edge_sc(Cy, Ce, Oe, pool, write_ea):
  Cpre = Cy + Oe
  Cout = Cpre // 3 if pool else Cpre
  Pw = Oe * Ce + Oe + Cy
  Pp = ((Pw + 15) // 16) * 16
  mesh = plsc.VectorSubcoreMesh(core_axis_name="c", subcore_axis_name="s")
  out_type = []
  if write_ea:
    out_type.append(jax.ShapeDtypeStruct((EP, Cout), _f32))
  out_type.append(jax.ShapeDtypeStruct((2, NP, Cout), _f32))
  scratch = [
      pltpu.VMEM((BCH, CHUNK), _i32),
      pltpu.VMEM((BCH, CHUNK), _i32),
      pltpu.VMEM((BLOCK * Ce,), _f32),
      pltpu.VMEM((BLOCK, Cy), _f32),
      pltpu.VMEM((BLOCK, Cy), _f32),
      pltpu.VMEM((BLOCK, Cout), _f32),
      pltpu.VMEM((Pp,), _f32),
      pltpu.VMEM_SHARED((NP, Cout), _f32),
      pltpu.SemaphoreType.DMA,
  ]

  def body(y_h, i0_h, i1_h, ea_h, w_h, z_h, *rest):
    if write_ea:
      eaout_h, agg_h = rest[0], rest[1]
      scr = rest[2:]
    else:
      agg_h = rest[0]
      scr = rest[1:]
    idx0, idx1, eab, r0, r1, outb, wvm, aggs, sem = scr
    cid = lax.axis_index("c")
    sid = lax.axis_index("s")
    wid = sid * 2 + cid

    pltpu.sync_copy(w_h, wvm)
    ws = []
    for k in range(Pp // 16):
      v = wvm[pl.ds(k * 16, 16)]
      ws.extend(v[j] for j in range(16))

    @pl.when(sid == 0)
    def _zero():
      pltpu.sync_copy(z_h, aggs)

    plsc.subcore_barrier()

    iota16 = lax.iota(_i32, 16)
    tile_ch = wid * CPT

    def block_body(blk, carry):
      ch0 = tile_ch + blk * BCH
      e0 = ch0 * CHUNK
      pltpu.sync_copy(i0_h.at[pl.ds(ch0, BCH)], idx0)
      pltpu.sync_copy(i1_h.at[pl.ds(ch0, BCH)], idx1)
      pltpu.sync_copy(ea_h.at[pl.ds(e0 * Ce, BLOCK * Ce)], eab)
      descs = []
      for k in range(BCH):
        descs.append(pltpu.async_copy(
            y_h.at[idx0.at[k]], r0.at[pl.ds(k * CHUNK, CHUNK)], sem))
        descs.append(pltpu.async_copy(
            y_h.at[idx1.at[k]], r1.at[pl.ds(k * CHUNK, CHUNK)], sem))
      for d in descs:
        d.wait()

      def group(g, c2):
        rows = g * 16 + iota16
        erows = rows * Ce
        hs = [jnp.zeros((16,), _f32) + ws[Oe * Ce + j] for j in range(Oe)]
        for c in range(Ce):
          a = plsc.load_gather(eab, [erows + c])
          for j in range(Oe):
            hs[j] = hs[j] + a * ws[j * Ce + c]

        def uch(i):
          if i >= Cy:
            return hs[i - Cy]
          a0 = plsc.load_gather(r0, [rows, jnp.full((16,), i, _i32)])
          a1 = plsc.load_gather(r1, [rows, jnp.full((16,), i, _i32)])
          return a0 + a1 + ws[Oe * Ce + Oe + i]

        for k in range(Cout):
          if pool:
            v = jnp.maximum(jnp.maximum(uch(3 * k), uch(3 * k + 1)),
                            uch(3 * k + 2))
          else:
            v = uch(k)
          v = jnp.maximum(v, 0.0)
          plsc.store_scatter(outb, [rows, jnp.full((16,), k, _i32)], v)
        return c2

      lax.fori_loop(0, BLOCK // 16, group, 0)
      if write_ea:
        pltpu.sync_copy(outb, eaout_h.at[pl.ds(e0, BLOCK)])
      for k in range(BCH):
        pltpu.sync_copy(outb.at[pl.ds(k * CHUNK, CHUNK)],
                        aggs.at[idx0.at[k]], add=True)
      return carry

    lax.fori_loop(0, NBLK, block_body, 0)
    plsc.subcore_barrier()

    @pl.when(sid == 0)
    def _flush():
      pltpu.sync_copy(aggs, agg_h.at[cid])

  return pl.kernel(
      body, out_type=out_type, mesh=mesh, scratch_types=scratch,
      compiler_params=pltpu.CompilerParams(needs_layout_passes=False,
                                           use_tc_tiling_on_sc=False))


def _tc_linear(xx, wt):
  n, ci = xx.shape
  o = wt.shape[1]

  def body(x_ref, w_ref, o_ref):
    o_ref[...] = jnp.dot(x_ref[...], w_ref[...],
                         preferred_element_type=_f32)

  return pl.pallas_call(
      body,
      grid=(n // BLKN,),
      in_specs=[pl.BlockSpec((BLKN, ci), lambda i: (i, 0)),
                pl.BlockSpec((ci, o), lambda i: (0, 0))],
      out_specs=pl.BlockSpec((BLKN, o), lambda i: (i, 0)),
      out_shape=jax.ShapeDtypeStruct((n, o), _f32),
  )(xx, wt)


def _tc_node(xx, a0, a1, wxt, bx, wet, be, wyt):
  n, ci = xx.shape
  ca = a0.shape[1]
  ga = wxt.shape[1] // 3
  gb = wet.shape[1] // 3
  cn = ga + gb
  oy = wyt.shape[1]

  def body(x_ref, a0_ref, a1_ref, wx_ref, bx_ref, we_ref, be_ref, wy_ref,
           ox_ref, oy_ref):
    xv = x_ref[...]
    agg = a0_ref[...] + a1_ref[...]
    xa = jnp.dot(xv, wx_ref[...], preferred_element_type=_f32) + bx_ref[...]
    xe = jnp.dot(agg, we_ref[...], preferred_element_type=_f32) + be_ref[...]
    xa = jnp.maximum(xa, 0.0)
    xe = jnp.maximum(xe, 0.0)
    ra = jnp.maximum(jnp.maximum(xa[:, :ga], xa[:, ga:2 * ga]),
                     xa[:, 2 * ga:])
    rb = jnp.maximum(jnp.maximum(xe[:, :gb], xe[:, gb:2 * gb]),
                     xe[:, 2 * gb:])
    xn = jnp.concatenate([ra, rb], axis=1)
    ox_ref[...] = xn
    oy_ref[...] = jnp.dot(xn, wy_ref[...], preferred_element_type=_f32)

  return pl.pallas_call(
      body,
      grid=(n // BLKN,),
      in_specs=[
          pl.BlockSpec((BLKN, ci), lambda i: (i, 0)),
          pl.BlockSpec((BLKN, ca), lambda i: (i, 0)),
          pl.BlockSpec((BLKN, ca), lambda i: (i, 0)),
          pl.BlockSpec(wxt.shape, lambda i: (0, 0)),
          pl.BlockSpec((1, 3 * ga), lambda i: (0, 0)),
          pl.BlockSpec(wet.shape, lambda i: (0, 0)),
          pl.BlockSpec((1, 3 * gb), lambda i: (0, 0)),
          pl.BlockSpec(wyt.shape, lambda i: (0, 0)),
      ],
      out_specs=[pl.BlockSpec((BLKN, cn), lambda i: (i, 0)),
                 pl.BlockSpec((BLKN, oy), lambda i: (i, 0))],
      out_shape=[jax.ShapeDtypeStruct((n, cn), _f32),
                 jax.ShapeDtypeStruct((n, oy), _f32)],
  )(xx, a0, a1, wxt, bx, wet, be, wyt)


def _tc_node5(xx, a0, a1, wxt, bx, wet, be):
  n, ci = xx.shape
  ca = a0.shape[1]
  ox = wxt.shape[1]
  oe = wet.shape[1]
  cn = ox + oe

  def body(x_ref, a0_ref, a1_ref, wx_ref, bx_ref, we_ref, be_ref,
           ox_ref, os_ref):
    i = pl.program_id(0)
    xv = x_ref[...]
    agg = a0_ref[...] + a1_ref[...]
    xa = jnp.maximum(
        jnp.dot(xv, wx_ref[...], preferred_element_type=_f32) + bx_ref[...],
        0.0)
    xe = jnp.maximum(
        jnp.dot(agg, we_ref[...], preferred_element_type=_f32) + be_ref[...],
        0.0)
    xn = jnp.concatenate([xa, xe], axis=1)
    ox_ref[...] = xn

    @pl.when(i == 0)
    def _():
      os_ref[...] = jnp.zeros_like(os_ref)

    os_ref[...] += jnp.sum(xn, axis=0, keepdims=True)

  return pl.pallas_call(
      body,
      grid=(n // BLKN,),
      in_specs=[
          pl.BlockSpec((BLKN, ci), lambda i: (i, 0)),
          pl.BlockSpec((BLKN, ca), lambda i: (i, 0)),
          pl.BlockSpec((BLKN, ca), lambda i: (i, 0)),
          pl.BlockSpec(wxt.shape, lambda i: (0, 0)),
          pl.BlockSpec((1, ox), lambda i: (0, 0)),
          pl.BlockSpec(wet.shape, lambda i: (0, 0)),
          pl.BlockSpec((1, oe), lambda i: (0, 0)),
      ],
      out_specs=[pl.BlockSpec((BLKN, cn), lambda i: (i, 0)),
                 pl.BlockSpec((1, cn), lambda i: (0, 0))],
      out_shape=[jax.ShapeDtypeStruct((n, cn), _f32),
                 jax.ShapeDtypeStruct((1, cn), _f32)],
  )(xx, a0, a1, wxt, bx, wet, be)


def _tc_mlp(xx, st, w1t, b1, w2t, b2, w3t, b3):
  n, c = xx.shape

  def body(x_ref, st_ref, w1_ref, b1_ref, w2_ref, b2_ref, w3_ref, b3_ref,
           o_ref):
    s = jnp.broadcast_to(st_ref[...], (BLKN, c))
    q = jnp.concatenate([s, x_ref[...]], axis=1)
    q = jnp.maximum(jnp.dot(q, w1_ref[...], preferred_element_type=_f32)
                    + b1_ref[...], 0.0)
    q = jnp.maximum(jnp.dot(q, w2_ref[...], preferred_element_type=_f32)
                    + b2_ref[...], 0.0)
    q = jnp.maximum(jnp.dot(q, w3_ref[...], preferred_element_type=_f32)
                    + b3_ref[...], 0.0)
    o_ref[...] = q

  return pl.pallas_call(
      body,
      grid=(n // BLKN,),
      in_specs=[
          pl.BlockSpec((BLKN, c), lambda i: (i, 0)),
          pl.BlockSpec((1, c), lambda i: (0, 0)),
          pl.BlockSpec(w1t.shape, lambda i: (0, 0)),
          pl.BlockSpec((1, w1t.shape[1]), lambda i: (0, 0)),
          pl.BlockSpec(w2t.shape, lambda i: (0, 0)),
          pl.BlockSpec((1, w2t.shape[1]), lambda i: (0, 0)),
          pl.BlockSpec(w3t.shape, lambda i: (0, 0)),
          pl.BlockSpec((1, w3t.shape[1]), lambda i: (0, 0)),
      ],
      out_specs=pl.BlockSpec((BLKN, 1), lambda i: (i, 0)),
      out_shape=jax.ShapeDtypeStruct((n, 1), _f32),
  )(xx, st, w1t, b1, w2t, b2, w3t, b3)


_EDGE_CFG = {
    'e1': (10, 1, 2, True, True),
    'e2': (18, 4, 12, True, True),
    'e3': (18, 10, 12, True, True),
    'e4': (18, 10, 12, True, True),
    'e5': (3, 10, 3, False, False),
}

_EDGE_KERNELS = {tag: _edge_sc(*cfg) for tag, cfg in _EDGE_CFG.items()}


def _pool_perm(o):
  g = o // 3
  return jnp.asarray([3 * (j % g) + j // g for j in range(o)], _i32)


def _pack_w(p, tag):
  cy, ce, oe, _, _ = _EDGE_CFG[tag]
  pw = oe * ce + oe + cy
  pp = ((pw + 15) // 16) * 16
  vec = jnp.concatenate([p[tag + 'e_w'].reshape(-1), p[tag + 'e_b'],
                         p[tag + 'x_b']])
  return jnp.concatenate([vec, jnp.zeros((pp - pw,), _f32)])


def _pad_y(y):
  return jnp.concatenate([y, jnp.zeros((NP - N_NODES, y.shape[1]), _f32)],
                         axis=0)


def kernel(x, edge_index, edge_attr, params):
  p = params
  x = x.astype(_f32)
  ei = edge_index.astype(_i32)
  e = ei.shape[1]
  pad = EP - e
  i0 = jnp.concatenate([ei[0], jnp.full((pad,), N_NODES, _i32)])
  i1 = jnp.concatenate([ei[1], jnp.full((pad,), N_NODES, _i32)])
  i0 = i0.reshape(EP // CHUNK, CHUNK)
  i1 = i1.reshape(EP // CHUNK, CHUNK)
  ea = jnp.concatenate([edge_attr.astype(_f32),
                        jnp.zeros((pad, edge_attr.shape[1]), _f32)], axis=0)

  def edge(tag, y, ea_in):
    cy, ce, oe, do_pool, write_ea = _EDGE_CFG[tag]
    cout = (cy + oe) // 3 if do_pool else (cy + oe)
    z = jnp.zeros((NP, cout), _f32)
    outs = _EDGE_KERNELS[tag](_pad_y(y), i0, i1, ea_in.reshape(-1),
                              _pack_w(p, tag), z)
    if not isinstance(outs, (list, tuple)):
      outs = (outs,)
    if write_ea:
      return outs[0], outs[1]
    return None, outs[0]

  def node(tag, xx, aggp, ynext_w):
    pa = _pool_perm(p[tag + 'x_w'].shape[0])
    pb = _pool_perm(p[tag + 'e_w'].shape[0])
    return _tc_node(
        xx, aggp[0, :N_NODES], aggp[1, :N_NODES],
        p[tag + 'x_w'][pa].T, p[tag + 'x_b'][pa].reshape(1, -1),
        p[tag + 'e_w'][pb].T, p[tag + 'e_b'][pb].reshape(1, -1),
        ynext_w.T)

  y = _tc_linear(x, p['e1x_w'].T)
  ea1, agg = edge('e1', y, ea)
  x1, y = node('n1', x, agg, p['e2x_w'])
  ea2, agg = edge('e2', y, ea1)
  x2, y = node('n2', x1, agg, p['e3x_w'])
  ea3, agg = edge('e3', y, ea2)
  x3, y = node('n3', x2, agg, p['e4x_w'])
  ea4, agg = edge('e4', y, ea3)
  x4, y = node('n4', x3, agg, p['e5x_w'])
  _, agg = edge('e5', y, ea4)
  x5, st = _tc_node5(
      x4, agg[0, :N_NODES], agg[1, :N_NODES],
      p['n5x_w'].T, p['n5x_b'].reshape(1, -1),
      p['n5e_w'].T, p['n5e_b'].reshape(1, -1))
  q = _tc_mlp(x5, st,
              p['fc1_w'].T, p['fc1_b'].reshape(1, -1),
              p['fc2_w'].T, p['fc2_b'].reshape(1, -1),
              p['fc3_w'].T, p['fc3_b'].reshape(1, -1))
  return q.reshape(-1)
